# baseline (speedup 1.0000x reference)
import functools

import jax
import jax.numpy as jnp
from jax import lax
from jax.experimental import pallas as pl
from jax.experimental.pallas import tpu as pltpu
from jax.experimental.pallas import tpu_sc as plsc

_LANES = 16


def _sc_gather_add(x_flat, idx, table):
    B, D = x_flat.shape
    info = plsc.get_sparse_core_info()
    NC, NS = info.num_cores, info.num_subcores
    NW = NC * NS
    b_per_w = B // NW
    K = 8
    KH = 4
    n_chunks = b_per_w // K
    n_loop = (n_chunks - 2) // 3 * 3
    NV = D // _LANES

    mesh = plsc.VectorSubcoreMesh(core_axis_name="c", subcore_axis_name="s")

    @functools.partial(
        pl.kernel,
        mesh=mesh,
        out_type=jax.ShapeDtypeStruct((B, D), jnp.float32),
        scratch_types=[
            pltpu.VMEM((b_per_w,), jnp.int32),
            pltpu.VMEM((K, D), jnp.float32),
            pltpu.VMEM((K, D), jnp.float32),
            pltpu.VMEM((K, D), jnp.float32),
            pltpu.VMEM((K, D), jnp.float32),
            pltpu.VMEM((K, D), jnp.float32),
            pltpu.VMEM((K, D), jnp.float32),
            pltpu.VMEM((KH, D), jnp.float32),
            pltpu.VMEM((KH, D), jnp.float32),
            pltpu.SemaphoreType.DMA,
            pltpu.SemaphoreType.DMA,
            pltpu.SemaphoreType.DMA,
            pltpu.SemaphoreType.DMA,
            pltpu.SemaphoreType.DMA,
            pltpu.SemaphoreType.DMA,
            pltpu.SemaphoreType.DMA,
            pltpu.SemaphoreType.DMA,
            pltpu.SemaphoreType.DMA,
        ],
    )
    def gather_add(x_hbm, idx_hbm, table_hbm, out_hbm, idx_v,
                   gbuf0, gbuf1, gbuf2, xbuf0, xbuf1, xbuf2, obuf0, obuf1,
                   gsem0, gsem1, gsem2, xsem0, xsem1, xsem2,
                   ssem0, ssem1, isem):
        gbufs, xbufs = (gbuf0, gbuf1, gbuf2), (xbuf0, xbuf1, xbuf2)
        gsems, xsems = (gsem0, gsem1, gsem2), (xsem0, xsem1, xsem2)
        obufs, ssems = (obuf0, obuf1), (ssem0, ssem1)

        wid = lax.axis_index("s") * NC + lax.axis_index("c")
        base = wid * b_per_w
        idx_dma = pltpu.async_copy(
            idx_hbm.at[pl.ds(base, b_per_w)], idx_v, isem)

        def issue_g(c, b):
            pltpu.async_copy(
                table_hbm.at[idx_v.at[pl.ds(c * K, K)]], gbufs[b], gsems[b])

        def issue_x(c, b):
            pltpu.async_copy(
                x_hbm.at[pl.ds(base + c * K, K)], xbufs[b], xsems[b])

        def wait_gx(b):
            pltpu.make_async_copy(
                table_hbm.at[idx_v.at[pl.ds(0, K)]], gbufs[b], gsems[b]).wait()
            pltpu.make_async_copy(
                x_hbm.at[pl.ds(0, K)], xbufs[b], xsems[b]).wait()

        def issue_store(c, h):
            pltpu.async_copy(
                obufs[h],
                out_hbm.at[pl.ds(base + c * K + h * KH, KH)], ssems[h])

        def wait_store(h):
            pltpu.make_async_copy(
                obufs[h], out_hbm.at[pl.ds(0, KH)], ssems[h]).wait()

        def halves(c, b, first_chunk):
            for h in (0, 1):
                if first_chunk:
                    pass
                else:
                    wait_store(h)

                def row_body(r, rc):
                    for j in range(NV):
                        sl = pl.ds(j * _LANES, _LANES)
                        obufs[h][r, sl] = (
                            gbufs[b][h * KH + r, sl] + xbufs[b][h * KH + r, sl])
                    return rc

                lax.fori_loop(0, KH, row_body, 0)
                issue_store(c, h)

        issue_x(0, 0)
        issue_x(1, 1)
        idx_dma.wait()
        issue_g(0, 0)
        issue_g(1, 1)

        wait_gx(0)
        issue_g(2, 2)
        issue_x(2, 2)
        halves(0, 0, first_chunk=True)

        def triple_body(c3, carry):
            for b in (1, 2, 0):
                c = 3 * c3 + (b if b else 3)
                pb = (b + 2) % 3
                wait_gx(b)

                @pl.when(c + 2 < n_chunks)
                def _prefetch():
                    issue_g(c + 2, pb)
                    issue_x(c + 2, pb)

                halves(c, b, first_chunk=False)
            return carry

        lax.fori_loop(0, (n_chunks - 1) // 3, triple_body, 0)

        c = n_chunks - 1
        b = c % 3
        wait_gx(b)
        halves(c, b, first_chunk=False)

        wait_store(0)
        wait_store(1)

    return gather_add(x_flat, idx, table)


def kernel(x, time_idx, emb_table):
    Bb, T, D = x.shape
    if T == time_idx.shape[1]:
        return x
    idx = time_idx[:, :T].reshape(-1).astype(jnp.int32)
    x_flat = x.reshape(Bb * T, D)
    out = _sc_gather_add(x_flat, idx, emb_table)
    return out.reshape(Bb, T, D)


# speedup vs baseline: 1.3575x; 1.3575x over previous
import functools

import jax
import jax.numpy as jnp
from jax import lax
from jax.experimental import pallas as pl
from jax.experimental.pallas import tpu as pltpu
from jax.experimental.pallas import tpu_sc as plsc

_LANES = 16


def _sc_gather_add(x_flat, idx, table):
    B, D = x_flat.shape
    info = plsc.get_sparse_core_info()
    NC, NS = info.num_cores, info.num_subcores
    NW = NC * NS
    b_per_w = B // NW
    K = 8
    n_chunks = b_per_w // K
    NV = D // _LANES

    mesh = plsc.VectorSubcoreMesh(core_axis_name="c", subcore_axis_name="s")

    @functools.partial(
        pl.kernel,
        mesh=mesh,
        out_type=jax.ShapeDtypeStruct((B, D), jnp.float32),
        scratch_types=[
            pltpu.VMEM((b_per_w,), jnp.int32),
            pltpu.VMEM((K, D), jnp.float32),
            pltpu.VMEM((K, D), jnp.float32),
            pltpu.VMEM((K, D), jnp.float32),
            pltpu.VMEM((K, D), jnp.float32),
            pltpu.VMEM((K, D), jnp.float32),
            pltpu.VMEM((K, D), jnp.float32),
            pltpu.SemaphoreType.DMA,
            pltpu.SemaphoreType.DMA,
            pltpu.SemaphoreType.DMA,
            pltpu.SemaphoreType.DMA,
            pltpu.SemaphoreType.DMA,
            pltpu.SemaphoreType.DMA,
            pltpu.SemaphoreType.DMA,
        ],
    )
    def gather_add(x_hbm, idx_hbm, table_hbm, out_hbm, idx_v,
                   gbuf0, gbuf1, xbuf0, xbuf1, obuf0, obuf1,
                   gsem0, gsem1, xsem0, xsem1, ssem0, ssem1, isem):
        gbufs, xbufs, obufs = (gbuf0, gbuf1), (xbuf0, xbuf1), (obuf0, obuf1)
        gsems, xsems, ssems = (gsem0, gsem1), (xsem0, xsem1), (ssem0, ssem1)

        wid = lax.axis_index("s") * NC + lax.axis_index("c")
        base = wid * b_per_w
        idx_dma = pltpu.async_copy(
            idx_hbm.at[pl.ds(base, b_per_w)], idx_v, isem)

        def issue_g(c, b):
            pltpu.async_copy(
                table_hbm.at[idx_v.at[pl.ds(c * K, K)]], gbufs[b], gsems[b])

        def issue_x(c, b):
            pltpu.async_copy(
                x_hbm.at[pl.ds(base + c * K, K)], xbufs[b], xsems[b])

        def issue_gx(c, b):
            issue_g(c, b)
            issue_x(c, b)

        def wait_gx(b):
            pltpu.make_async_copy(
                table_hbm.at[idx_v.at[pl.ds(0, K)]], gbufs[b], gsems[b]).wait()
            pltpu.make_async_copy(
                x_hbm.at[pl.ds(0, K)], xbufs[b], xsems[b]).wait()

        def issue_store(c, b):
            pltpu.async_copy(
                obufs[b], out_hbm.at[pl.ds(base + c * K, K)], ssems[b])

        def wait_store(b):
            pltpu.make_async_copy(
                obufs[b], out_hbm.at[pl.ds(0, K)], ssems[b]).wait()

        issue_x(0, 0)
        issue_x(1, 1)
        idx_dma.wait()
        issue_g(0, 0)
        issue_g(1, 1)

        def pair_body(c2, carry):
            for b in (0, 1):
                c = 2 * c2 + b
                wait_gx(b)

                @pl.when(c >= 2)
                def _drain():
                    wait_store(b)

                def row_body(r, rc):
                    for j in range(NV):
                        sl = pl.ds(j * _LANES, _LANES)
                        obufs[b][r, sl] = gbufs[b][r, sl] + xbufs[b][r, sl]
                    return rc

                lax.fori_loop(0, K, row_body, 0)
                issue_store(c, b)

                @pl.when(c + 2 < n_chunks)
                def _prefetch():
                    issue_gx(c + 2, b)
            return carry

        lax.fori_loop(0, n_chunks // 2, pair_body, 0)
        wait_store(0)
        wait_store(1)

    return gather_add(x_flat, idx, table)


def kernel(x, time_idx, emb_table):
    Bb, T, D = x.shape
    if T == time_idx.shape[1]:
        return x
    idx = time_idx[:, :T].reshape(-1).astype(jnp.int32)
    x_flat = x.reshape(Bb * T, D)
    out = _sc_gather_add(x_flat, idx, emb_table)
    return out.reshape(Bb, T, D)
